# baseline (speedup 1.0000x reference)
import functools

import jax
import jax.numpy as jnp
from jax import lax
from jax.experimental import pallas as pl
from jax.experimental.pallas import tpu as pltpu
from jax.experimental.pallas import tpu_sc as plsc

EDIM = 64
NFIELDS = 7
D_MODEL = 512

_NC = 2
_NS = 16
_NW = _NC * _NS
_CHUNK = 128


def _gather_body(xt, t0, t1, t2, t3, t4, t5, t6, out, idx_v, rows_v, sem):
    n_chunks = xt.shape[1]
    cpw = n_chunks // _NW
    tpw = cpw * _CHUNK
    wid = lax.axis_index("s") * _NC + lax.axis_index("c")
    base = wid * tpw
    tables = [t0, t1, t2, t3, t4, t5, t6]
    for i in range(NFIELDS):
        pltpu.sync_copy(xt.at[i, pl.ds(wid * cpw, cpw)], idx_v)
        for c in range(cpw):
            pltpu.async_copy(
                tables[i].at[idx_v.at[c]],
                rows_v.at[pl.ds(c * _CHUNK, _CHUNK)],
                sem,
            ).wait()
        pltpu.sync_copy(rows_v, out.at[i, pl.ds(base, tpw)])


@functools.cache
def _make_gather(n_tokens):
    tpw = n_tokens // _NW
    mesh = plsc.VectorSubcoreMesh(core_axis_name="c", subcore_axis_name="s")
    return functools.partial(
        pl.kernel,
        out_type=jax.ShapeDtypeStruct((NFIELDS, n_tokens, EDIM), jnp.float32),
        mesh=mesh,
        scratch_types=[
            pltpu.VMEM((tpw // _CHUNK, _CHUNK), jnp.int32),
            pltpu.VMEM((tpw, EDIM), jnp.float32),
            pltpu.SemaphoreType.DMA,
        ],
        compiler_params=pltpu.CompilerParams(use_tc_tiling_on_sc=False),
    )(_gather_body)


def _mm_body(h_ref, w_ref, b_ref, o_ref):
    acc = b_ref[...].astype(jnp.float32)
    for i in range(NFIELDS):
        acc = acc + jnp.dot(
            h_ref[i], w_ref[i], preferred_element_type=jnp.float32
        )
    o_ref[...] = acc


@functools.cache
def _make_matmul(n_tokens, tm):
    return pl.pallas_call(
        _mm_body,
        grid=(n_tokens // tm,),
        in_specs=[
            pl.BlockSpec((NFIELDS, tm, EDIM), lambda m: (0, m, 0)),
            pl.BlockSpec((NFIELDS, EDIM, D_MODEL), lambda m: (0, 0, 0)),
            pl.BlockSpec((1, D_MODEL), lambda m: (0, 0)),
        ],
        out_specs=pl.BlockSpec((tm, D_MODEL), lambda m: (m, 0)),
        out_shape=jax.ShapeDtypeStruct((n_tokens, D_MODEL), jnp.float32),
    )


def kernel(x, T0, T1, T2, T3, T4, T5, T6, W, b):
    bsz, seq, nf = x.shape
    n = bsz * seq
    xt = (
        x.reshape(n, NFIELDS)
        .T.astype(jnp.int32)
        .reshape(NFIELDS, n // _CHUNK, _CHUNK)
    )
    h = _make_gather(n)(xt, T0, T1, T2, T3, T4, T5, T6)
    wt = W.T.reshape(NFIELDS, EDIM, D_MODEL)
    out = _make_matmul(n, 512)(h, wt, b.reshape(1, D_MODEL))
    return out.reshape(bsz, seq, D_MODEL)


# speedup vs baseline: 1.0279x; 1.0279x over previous
import functools

import jax
import jax.numpy as jnp
from jax import lax
from jax.experimental import pallas as pl
from jax.experimental.pallas import tpu as pltpu
from jax.experimental.pallas import tpu_sc as plsc

EDIM = 64
NFIELDS = 7
D_MODEL = 512

_NC = 2
_NS = 16
_NW = _NC * _NS
_CHUNK = 128
_TPW = 256
_CPW = _TPW // _CHUNK


def _gather_body(xt, t0, t1, t2, t3, t4, t5, t6, out, idx_v, rows_v, gsem,
                 wsem0, wsem1):
    wid = lax.axis_index("s") * _NC + lax.axis_index("c")
    base = wid * _TPW
    tables = [t0, t1, t2, t3, t4, t5, t6]
    wsems = [wsem0, wsem1]
    pltpu.sync_copy(xt.at[wid], idx_v)
    wb = [None, None]
    for i in range(NFIELDS):
        s = i % 2
        if wb[s] is not None:
            wb[s].wait()
        g = [
            pltpu.async_copy(
                tables[i].at[idx_v.at[_CPW * i + c]],
                rows_v.at[s, pl.ds(c * _CHUNK, _CHUNK)],
                gsem,
            )
            for c in range(_CPW)
        ]
        for cp in g:
            cp.wait()
        wb[s] = pltpu.async_copy(rows_v.at[s], out.at[i, pl.ds(base, _TPW)],
                                 wsems[s])
    wb[0].wait()
    wb[1].wait()


@functools.cache
def _make_gather(n_tokens):
    mesh = plsc.VectorSubcoreMesh(core_axis_name="c", subcore_axis_name="s")
    return functools.partial(
        pl.kernel,
        out_type=jax.ShapeDtypeStruct((NFIELDS, n_tokens, 2 * EDIM),
                                      jnp.float32),
        mesh=mesh,
        scratch_types=[
            pltpu.VMEM((NFIELDS * _CPW, _CHUNK), jnp.int32),
            pltpu.VMEM((2, _TPW, 2 * EDIM), jnp.float32),
            pltpu.SemaphoreType.DMA,
            pltpu.SemaphoreType.DMA,
            pltpu.SemaphoreType.DMA,
        ],
    )(_gather_body)


def _mm_body(h_ref, m_ref, w_ref, b_ref, o_ref):
    acc = b_ref[...].astype(jnp.float32)
    for i in range(NFIELDS):
        wide = h_ref[i]
        ev = wide[:, :EDIM]
        od = wide[:, EDIM:]
        m = m_ref[i].reshape(wide.shape[0], 1)
        h_i = ev + m * (od - ev)
        acc = acc + jnp.dot(h_i, w_ref[i], preferred_element_type=jnp.float32)
    o_ref[...] = acc


@functools.cache
def _make_matmul(n_tokens, tm):
    return pl.pallas_call(
        _mm_body,
        grid=(n_tokens // tm,),
        in_specs=[
            pl.BlockSpec((NFIELDS, tm, 2 * EDIM), lambda m: (0, m, 0)),
            pl.BlockSpec((NFIELDS, tm), lambda m: (0, m)),
            pl.BlockSpec((NFIELDS, EDIM, D_MODEL), lambda m: (0, 0, 0)),
            pl.BlockSpec((1, D_MODEL), lambda m: (0, 0)),
        ],
        out_specs=pl.BlockSpec((tm, D_MODEL), lambda m: (m, 0)),
        out_shape=jax.ShapeDtypeStruct((n_tokens, D_MODEL), jnp.float32),
    )


def kernel(x, T0, T1, T2, T3, T4, T5, T6, W, b):
    bsz, seq, nf = x.shape
    n = bsz * seq
    xr = x.reshape(n, NFIELDS).astype(jnp.int32)
    xt = (
        (xr >> 1)
        .T.reshape(NFIELDS, _NW, _CPW, _CHUNK)
        .transpose(1, 0, 2, 3)
        .reshape(_NW, NFIELDS * _CPW, _CHUNK)
    )
    m = (xr & 1).astype(jnp.float32).T
    tables = [T.reshape(T.shape[0] // 2, 2 * EDIM)
              for T in (T0, T1, T2, T3, T4, T5, T6)]
    h = _make_gather(n)(xt, *tables)
    wt = W.T.reshape(NFIELDS, EDIM, D_MODEL)
    out = _make_matmul(n, 512)(h, m, wt, b.reshape(1, D_MODEL))
    return out.reshape(bsz, seq, D_MODEL)


# speedup vs baseline: 1.4918x; 1.4513x over previous
import functools

import jax
import jax.numpy as jnp
from jax import lax
from jax.experimental import pallas as pl
from jax.experimental.pallas import tpu as pltpu
from jax.experimental.pallas import tpu_sc as plsc

EDIM = 64
NFIELDS = 7
D_MODEL = 512

_NC = 2
_NS = 16
_NW = _NC * _NS
_CHUNK = 128
_TPW = 256
_BK = 4096
_HB = _BK // 2

_SC_PARAMS = pltpu.CompilerParams(use_tc_tiling_on_sc=True)


def _tr_body(t_ref, o_ref):
    x = t_ref[...]
    o_ref[:, :EDIM] = x[:, :_HB].T
    o_ref[:, EDIM:] = x[:, _HB:].T


@functools.cache
def _make_transpose(vocab):
    grid = (vocab + _BK - 1) // _BK
    return pl.pallas_call(
        _tr_body,
        grid=(grid,),
        in_specs=[pl.BlockSpec((EDIM, _BK), lambda g: (0, g))],
        out_specs=pl.BlockSpec((_HB, 2 * EDIM), lambda g: (g, 0)),
        out_shape=jax.ShapeDtypeStruct((grid * _HB, 2 * EDIM), jnp.float32),
    )


def _gather_body(xti, t2, out, idx_v, rows_v, gsem):
    wid = lax.axis_index("s") * _NC + lax.axis_index("c")
    base = wid * _TPW
    pltpu.sync_copy(xti.at[wid], idx_v)
    gs = [
        pltpu.async_copy(
            t2.at[idx_v.at[c]],
            rows_v.at[pl.ds(c * _CHUNK, _CHUNK)],
            gsem,
        )
        for c in range(_TPW // _CHUNK)
    ]
    for g in gs:
        g.wait()
    pltpu.sync_copy(rows_v, out.at[pl.ds(base, _TPW)])


@functools.cache
def _make_gather(n_tokens, n_lines):
    mesh = plsc.VectorSubcoreMesh(core_axis_name="c", subcore_axis_name="s")
    return functools.partial(
        pl.kernel,
        out_type=jax.ShapeDtypeStruct((n_tokens, 2 * EDIM), jnp.float32),
        mesh=mesh,
        scratch_types=[
            pltpu.VMEM((8, _CHUNK), jnp.int32),
            pltpu.VMEM((_TPW, 2 * EDIM), jnp.float32),
            pltpu.SemaphoreType.DMA,
        ],
        compiler_params=_SC_PARAMS,
    )(_gather_body)


def _mm_body(h0, h1, h2, h3, h4, h5, h6, m_ref, w_ref, b_ref, o_ref):
    acc = b_ref[...].astype(jnp.float32)
    tm = o_ref.shape[0]
    for i, h_ref in enumerate((h0, h1, h2, h3, h4, h5, h6)):
        wide = h_ref[...]
        ev = wide[:, :EDIM]
        od = wide[:, EDIM:]
        sel = m_ref[i].reshape(tm, 1) > 0.5
        h_i = jnp.where(sel, od, ev)
        acc = acc + jnp.dot(h_i, w_ref[i], preferred_element_type=jnp.float32)
    o_ref[...] = acc


@functools.cache
def _make_matmul(n_tokens, tm):
    h_spec = pl.BlockSpec((tm, 2 * EDIM), lambda m: (m, 0))
    return pl.pallas_call(
        _mm_body,
        grid=(n_tokens // tm,),
        in_specs=[h_spec] * NFIELDS + [
            pl.BlockSpec((NFIELDS, tm), lambda m: (0, m)),
            pl.BlockSpec((NFIELDS, EDIM, D_MODEL), lambda m: (0, 0, 0)),
            pl.BlockSpec((1, D_MODEL), lambda m: (0, 0)),
        ],
        out_specs=pl.BlockSpec((tm, D_MODEL), lambda m: (m, 0)),
        out_shape=jax.ShapeDtypeStruct((n_tokens, D_MODEL), jnp.float32),
    )


def kernel(x, T0, T1, T2, T3, T4, T5, T6, W, b):
    bsz, seq, nf = x.shape
    n = bsz * seq
    xr = x.reshape(n, NFIELDS).astype(jnp.int32)
    line = (xr // _BK) * _HB + (xr & (_HB - 1))
    half = ((xr // _HB) & 1).astype(jnp.float32)
    xt = jnp.pad(
        line.T.reshape(NFIELDS, _NW, _TPW // _CHUNK, _CHUNK),
        ((0, 0), (0, 0), (0, 8 - _TPW // _CHUNK), (0, 0)),
    )
    m = half.T
    hs = []
    for i, T in enumerate((T0, T1, T2, T3, T4, T5, T6)):
        t2 = _make_transpose(T.shape[0])(jnp.transpose(T))
        hs.append(_make_gather(n, t2.shape[0])(xt[i], t2))
    wt = W.T.reshape(NFIELDS, EDIM, D_MODEL)
    out = _make_matmul(n, 512)(*hs, m, wt, b.reshape(1, D_MODEL))
    return out.reshape(bsz, seq, D_MODEL)
